# Initial kernel scaffold; baseline (speedup 1.0000x reference)
import jax
import jax.numpy as jnp
from jax.experimental import pallas as pl


def kernel(x, edge_index, edge_weight):
    raise NotImplementedError("write your pallas kernel here")


# speedup vs baseline: 14.3564x; 14.3564x over previous
import functools

import jax
import jax.numpy as jnp
from jax import lax
from jax.experimental import pallas as pl
from jax.experimental.pallas import tpu as pltpu
from jax.experimental.pallas import tpu_sc as plsc

N = 10000
E = 320000
D = 128
ALPHA_C = 0.3

NC, NS = 2, 16
NT = NC * NS
EPT = E // NT
K = 128
C = 10240 // K
PAD = C * K - EPT
NA = 10240
RPT = NA // NS


def _mesh():
    return plsc.VectorSubcoreMesh(core_axis_name="c", subcore_axis_name="s",
                                  num_cores=NC, num_subcores=NS)


@functools.partial(
    pl.kernel,
    out_type=jax.ShapeDtypeStruct((NT, N), jnp.float32),
    mesh=_mesh(),
    scratch_types=[
        pltpu.VMEM((C, K), jnp.int32),
        pltpu.VMEM((C * K,), jnp.float32),
        pltpu.VMEM((N,), jnp.float32),
    ],
    compiler_params=pltpu.CompilerParams(needs_layout_passes=False),
)
def _deg_kernel(col_hbm, w_hbm, out_hbm, col_v, w_v, acc):
    cid = lax.axis_index("c")
    sid = lax.axis_index("s")
    t = cid * NS + sid

    def _z(i, carry):
        acc[pl.ds(i * 16, 16)] = jnp.zeros((16,), jnp.float32)
        return carry
    lax.fori_loop(0, N // 16, _z, 0)

    pltpu.sync_copy(col_hbm.at[t], col_v)
    pltpu.sync_copy(w_hbm.at[t], w_v)

    def _grp(g, carry):
        j = g // 8
        f = g % 8
        idx = col_v[j, pl.ds(f * 16, 16)]
        wv = w_v[pl.ds(g * 16, 16)]
        plsc.addupdate_scatter(acc, [idx], wv)
        return carry
    lax.fori_loop(0, C * 8, _grp, 0)

    pltpu.sync_copy(acc, out_hbm.at[t])


@functools.partial(
    pl.kernel,
    out_type=jax.ShapeDtypeStruct((NC, NA, D), jnp.float32),
    mesh=_mesh(),
    scratch_types=[
        pltpu.VMEM((C, K), jnp.int32),
        pltpu.VMEM((C, K), jnp.int32),
        pltpu.VMEM((C * K,), jnp.float32),
        pltpu.VMEM((K, D), jnp.float32),
        pltpu.VMEM_SHARED((NA, D), jnp.float32),
    ],
    compiler_params=pltpu.CompilerParams(needs_layout_passes=False),
)
def _edge_kernel(xs_hbm, row_hbm, col_hbm, w_hbm, out_hbm,
                 row_v, col_v, w_v, buf, t_sp):
    cid = lax.axis_index("c")
    sid = lax.axis_index("s")
    t = cid * NS + sid
    base = sid * RPT

    def _zrow(i, carry):
        for f in range(8):
            buf[i, pl.ds(f * 16, 16)] = jnp.zeros((16,), jnp.float32)
        return carry
    lax.fori_loop(0, K, _zrow, 0)
    for i in range(5):
        pltpu.sync_copy(buf, t_sp.at[pl.ds(base + i * K, K)])
    plsc.subcore_barrier()

    pltpu.sync_copy(row_hbm.at[t], row_v)
    pltpu.sync_copy(col_hbm.at[t], col_v)
    pltpu.sync_copy(w_hbm.at[t], w_v)

    def _chunk(j, carry):
        pltpu.sync_copy(xs_hbm.at[row_v.at[j]], buf)

        def _scale(kk, c2):
            wv = plsc.load_gather(
                w_v, [jnp.full((16,), j * K + kk, jnp.int32)])
            for f in range(8):
                sl = pl.ds(f * 16, 16)
                buf[kk, sl] = buf[kk, sl] * wv
            return c2
        lax.fori_loop(0, K, _scale, 0)

        pltpu.sync_copy(buf, t_sp.at[col_v.at[j]], add=True)
        return carry
    lax.fori_loop(0, C, _chunk, 0)

    plsc.subcore_barrier()
    pltpu.sync_copy(t_sp.at[pl.ds(base, RPT)],
                    out_hbm.at[cid, pl.ds(base, RPT)])


def _prep_body(dp_ref, x_ref, xs_ref, disb_ref):
    deg = 1.0 + jnp.sum(dp_ref[...], axis=0)[:, None]
    dis = lax.rsqrt(deg)
    xs_ref[...] = dis * x_ref[...]
    disb_ref[...] = jnp.broadcast_to(dis, (N, D))


_prep = pl.pallas_call(
    _prep_body,
    out_shape=[
        jax.ShapeDtypeStruct((N, D), jnp.float32),
        jax.ShapeDtypeStruct((N, D), jnp.float32),
    ],
)


def _combine_body(tp_ref, h_ref, disb_ref, out_ref, xs_ref):
    dis = disb_ref[...]
    h = h_ref[...]
    t_sum = tp_ref[0, :N, :] + tp_ref[1, :N, :]
    agg = dis * t_sum + dis * dis * h
    o = jax.nn.sigmoid((1.0 - ALPHA_C) * agg + ALPHA_C * h)
    out_ref[...] = o
    xs_ref[...] = dis * o


_combine = pl.pallas_call(
    _combine_body,
    out_shape=[
        jax.ShapeDtypeStruct((N, D), jnp.float32),
        jax.ShapeDtypeStruct((N, D), jnp.float32),
    ],
)


def _prepare_edges(edge_index, edge_weight):
    row = edge_index[0].astype(jnp.int32).reshape(NT, EPT)
    col = edge_index[1].astype(jnp.int32).reshape(NT, EPT)
    w = edge_weight.astype(jnp.float32).reshape(NT, EPT)

    spread = (jnp.arange(PAD, dtype=jnp.int32) * 41) % N
    pad_idx = jnp.broadcast_to(spread, (NT, PAD))
    pad_w = jnp.zeros((NT, PAD), jnp.float32)
    row_p = jnp.concatenate([row, pad_idx], axis=1).reshape(NT, C, K)
    col_p = jnp.concatenate([col, pad_idx], axis=1).reshape(NT, C, K)
    w_p = jnp.concatenate([w, pad_w], axis=1)
    return row_p, col_p, w_p


def kernel(x, edge_index, edge_weight):
    x = x.astype(jnp.float32)
    row_p, col_p, w_p = _prepare_edges(edge_index, edge_weight)

    deg_parts = _deg_kernel(col_p, w_p)
    xs1, disb = _prep(deg_parts, x)
    t_parts = _edge_kernel(xs1, row_p, col_p, w_p)
    h, xs2 = _combine(t_parts, x, disb)
    u_parts = _edge_kernel(xs2, row_p, col_p, w_p)
    out, _ = _combine(u_parts, h, disb)
    return out


# speedup vs baseline: 25.2315x; 1.7575x over previous
import functools

import jax
import jax.numpy as jnp
from jax import lax
from jax.experimental import pallas as pl
from jax.experimental.pallas import tpu as pltpu
from jax.experimental.pallas import tpu_sc as plsc

N = 10000
E = 320000
D = 128
ALPHA_C = 0.3

NC, NS = 2, 16
NT = NC * NS
EPT = E // NT
K = 112
C = 93
CK = C * K
PAD = CK - EPT
NA = 10112
RPT = NA // NS
NBUF = 3
GPC = C // NBUF


def _mesh():
    return plsc.VectorSubcoreMesh(core_axis_name="c", subcore_axis_name="s",
                                  num_cores=NC, num_subcores=NS)


@functools.partial(
    pl.kernel,
    out_type=jax.ShapeDtypeStruct((NT, N), jnp.float32),
    mesh=_mesh(),
    scratch_types=[
        pltpu.VMEM((CK,), jnp.int32),
        pltpu.VMEM((CK,), jnp.float32),
        pltpu.VMEM((N,), jnp.float32),
    ],
    compiler_params=pltpu.CompilerParams(needs_layout_passes=False),
)
def _deg_kernel(col_hbm, w_hbm, out_hbm, col_v, w_v, acc):
    cid = lax.axis_index("c")
    sid = lax.axis_index("s")
    t = cid * NS + sid

    def _z(i, carry):
        acc[pl.ds(i * 16, 16)] = jnp.zeros((16,), jnp.float32)
        return carry
    lax.fori_loop(0, N // 16, _z, 0)

    pltpu.sync_copy(col_hbm.at[t], col_v)
    pltpu.sync_copy(w_hbm.at[t], w_v)

    def _grp(g, carry):
        idx = col_v[pl.ds(g * 16, 16)]
        wv = w_v[pl.ds(g * 16, 16)]
        plsc.addupdate_scatter(acc, [idx], wv)
        return carry
    lax.fori_loop(0, CK // 16, _grp, 0)

    pltpu.sync_copy(acc, out_hbm.at[t])


@functools.partial(
    pl.kernel,
    out_type=jax.ShapeDtypeStruct((NC, NA, D), jnp.float32),
    mesh=_mesh(),
    scratch_types=(
        [pltpu.VMEM((K, D), jnp.float32)] * NBUF
        + [pltpu.VMEM((K,), jnp.int32)] * NBUF
        + [pltpu.VMEM((K,), jnp.int32)] * NBUF
        + [pltpu.VMEM((K,), jnp.float32)] * NBUF
        + [pltpu.VMEM_SHARED((NA, D), jnp.float32)]
        + [pltpu.SemaphoreType.DMA] * (4 * NBUF)
    ),
    compiler_params=pltpu.CompilerParams(needs_layout_passes=False),
)
def _edge_kernel(xs_hbm, rc_hbm, w_hbm, out_hbm,
                 b0, b1, b2, rc0, rc1, rc2, ci0, ci1, ci2, w0, w1, w2,
                 t_sp,
                 gs0, gs1, gs2, ss0, ss1, ss2,
                 rs0, rs1, rs2, ws0, ws1, ws2):
    bufs = (b0, b1, b2)
    rcs = (rc0, rc1, rc2)
    cis = (ci0, ci1, ci2)
    wvs = (w0, w1, w2)
    gsems = (gs0, gs1, gs2)
    ssems = (ss0, ss1, ss2)
    rsems = (rs0, rs1, rs2)
    wsems = (ws0, ws1, ws2)

    cid = lax.axis_index("c")
    sid = lax.axis_index("s")
    t = cid * NS + sid
    base = sid * RPT

    def _zrow(i, carry):
        for f in range(8):
            b0[i, pl.ds(f * 16, 16)] = jnp.zeros((16,), jnp.float32)
        return carry
    lax.fori_loop(0, K, _zrow, 0)
    for i in range(5):
        pltpu.sync_copy(b0, t_sp.at[pl.ds(base + i * K, K)])
    pltpu.sync_copy(b0.at[pl.ds(0, RPT - 5 * K)],
                    t_sp.at[pl.ds(base + 5 * K, RPT - 5 * K)])
    plsc.subcore_barrier()

    def _start_stage(j, s):
        pltpu.async_copy(rc_hbm.at[t, j], rcs[s], rsems[s])
        pltpu.async_copy(w_hbm.at[t, j], wvs[s], wsems[s])

    def _wait_stage(j, s):
        pltpu.make_async_copy(rc_hbm.at[t, j], rcs[s], rsems[s]).wait()
        pltpu.make_async_copy(w_hbm.at[t, j], wvs[s], wsems[s]).wait()

    def _unpack(s):
        for g in range(K // 16):
            sl = pl.ds(g * 16, 16)
            v = rcs[s][sl]
            rcs[s][sl] = v & 0x3FFF
            cis[s][sl] = lax.shift_right_logical(v, 14)

    def _start_gather(j, s):
        pltpu.async_copy(xs_hbm.at[rcs[s]], bufs[s], gsems[s])

    def _wait_gather(j, s):
        pltpu.make_async_copy(xs_hbm.at[rcs[s]], bufs[s], gsems[s]).wait()

    def _start_scatter(j, s):
        pltpu.async_copy(bufs[s], t_sp.at[cis[s]], ssems[s], add=True)

    def _wait_scatter(j, s):
        pltpu.make_async_copy(bufs[s], t_sp.at[cis[s]], ssems[s]).wait()

    _start_stage(0, 0)
    _start_stage(1, 1)
    _wait_stage(0, 0)
    _unpack(0)
    _start_gather(0, 0)

    def _group(m, carry):
        for i in range(NBUF):
            j = m * NBUF + i
            s = i
            n = (i + 1) % NBUF
            p = (i + 2) % NBUF

            _wait_gather(j, s)

            def _stage_next():
                _start_stage(j + 2, p)
            if i == 0:
                _stage_next()
            else:
                pl.when(m < GPC - 1)(_stage_next)

            def _prep_next():
                _wait_scatter(j - 2, n)

            def _launch_next():
                _wait_stage(j + 1, n)
                _unpack(n)
                _start_gather(j + 1, n)

            if i == 2:
                _prep_next()
                pl.when(m < GPC - 1)(_launch_next)
            else:
                pl.when(m > 0)(_prep_next)
                _launch_next()

            b = bufs[s]
            wr = wvs[s]

            @plsc.parallel_loop(0, K, unroll=8)
            def _scale(kk):
                wv = plsc.load_gather(wr, [jnp.full((16,), kk, jnp.int32)])
                for f in range(8):
                    sl = pl.ds(f * 16, 16)
                    b[kk, sl] = b[kk, sl] * wv

            _start_scatter(j, s)
        return carry
    lax.fori_loop(0, GPC, _group, 0)

    _wait_scatter(C - 2, (C - 2) % NBUF)
    _wait_scatter(C - 1, (C - 1) % NBUF)

    plsc.subcore_barrier()
    pltpu.sync_copy(t_sp.at[pl.ds(base, RPT)],
                    out_hbm.at[cid, pl.ds(base, RPT)])


def _prep_body(dp_ref, x_ref, xs_ref, disb_ref):
    deg = 1.0 + jnp.sum(dp_ref[...], axis=0)[:, None]
    dis = lax.rsqrt(deg)
    xs_ref[...] = dis * x_ref[...]
    disb_ref[...] = jnp.broadcast_to(dis, (N, D))


_prep = pl.pallas_call(
    _prep_body,
    out_shape=[
        jax.ShapeDtypeStruct((N, D), jnp.float32),
        jax.ShapeDtypeStruct((N, D), jnp.float32),
    ],
)


def _combine_body(tp_ref, h_ref, disb_ref, out_ref, xs_ref):
    dis = disb_ref[...]
    h = h_ref[...]
    t_sum = tp_ref[0, :N, :] + tp_ref[1, :N, :]
    agg = dis * t_sum + dis * dis * h
    o = jax.nn.sigmoid((1.0 - ALPHA_C) * agg + ALPHA_C * h)
    out_ref[...] = o
    xs_ref[...] = dis * o


_combine = pl.pallas_call(
    _combine_body,
    out_shape=[
        jax.ShapeDtypeStruct((N, D), jnp.float32),
        jax.ShapeDtypeStruct((N, D), jnp.float32),
    ],
)


def _prepare_edges(edge_index, edge_weight):
    row = edge_index[0].astype(jnp.int32).reshape(NT, EPT)
    col = edge_index[1].astype(jnp.int32).reshape(NT, EPT)
    w = edge_weight.astype(jnp.float32).reshape(NT, EPT)

    spread = (jnp.arange(PAD, dtype=jnp.int32) * 41) % N
    pad_idx = jnp.broadcast_to(spread, (NT, PAD))
    pad_w = jnp.zeros((NT, PAD), jnp.float32)
    row_p = jnp.concatenate([row, pad_idx], axis=1)
    col_p = jnp.concatenate([col, pad_idx], axis=1)
    w_p = jnp.concatenate([w, pad_w], axis=1)
    rc_p = (row_p | (col_p << 14)).reshape(NT, C, K)
    return rc_p, col_p, w_p


def kernel(x, edge_index, edge_weight):
    x = x.astype(jnp.float32)
    rc_p, col_p, w_p = _prepare_edges(edge_index, edge_weight)
    w_ck = w_p.reshape(NT, C, K)

    deg_parts = _deg_kernel(col_p, w_p)
    xs1, disb = _prep(deg_parts, x)
    t_parts = _edge_kernel(xs1, rc_p, w_ck)
    h, xs2 = _combine(t_parts, x, disb)
    u_parts = _edge_kernel(xs2, rc_p, w_ck)
    out, _ = _combine(u_parts, h, disb)
    return out


# speedup vs baseline: 25.7196x; 1.0193x over previous
import functools

import jax
import jax.numpy as jnp
from jax import lax
from jax.experimental import pallas as pl
from jax.experimental.pallas import tpu as pltpu
from jax.experimental.pallas import tpu_sc as plsc

N = 10000
E = 320000
D = 128
ALPHA_C = 0.3

NC, NS = 2, 16
NT = NC * NS
EPT = E // NT
K = 112
C = 93
CK = C * K
PAD = CK - EPT
NA = 10112
RPT = NA // NS
NBUF = 3
GPC = C // NBUF


def _mesh():
    return plsc.VectorSubcoreMesh(core_axis_name="c", subcore_axis_name="s",
                                  num_cores=NC, num_subcores=NS)


@functools.partial(
    pl.kernel,
    out_type=jax.ShapeDtypeStruct((NT, N), jnp.float32),
    mesh=_mesh(),
    scratch_types=[
        pltpu.VMEM((CK,), jnp.int32),
        pltpu.VMEM((CK,), jnp.float32),
        pltpu.VMEM((N,), jnp.float32),
    ],
    compiler_params=pltpu.CompilerParams(needs_layout_passes=False),
)
def _deg_kernel(col_hbm, w_hbm, out_hbm, col_v, w_v, acc):
    cid = lax.axis_index("c")
    sid = lax.axis_index("s")
    t = cid * NS + sid

    def _z(i, carry):
        acc[pl.ds(i * 16, 16)] = jnp.zeros((16,), jnp.float32)
        return carry
    lax.fori_loop(0, N // 16, _z, 0)

    pltpu.sync_copy(col_hbm.at[t], col_v)
    pltpu.sync_copy(w_hbm.at[t], w_v)

    def _grp(g, carry):
        idx = col_v[pl.ds(g * 16, 16)]
        wv = w_v[pl.ds(g * 16, 16)]
        plsc.addupdate_scatter(acc, [idx], wv)
        return carry
    lax.fori_loop(0, CK // 16, _grp, 0)

    pltpu.sync_copy(acc, out_hbm.at[t])


@functools.partial(
    pl.kernel,
    out_type=jax.ShapeDtypeStruct((NC, NA, D), jnp.float32),
    mesh=_mesh(),
    scratch_types=(
        [pltpu.VMEM((K, D), jnp.float32)] * NBUF
        + [pltpu.VMEM((K,), jnp.int32)] * NBUF
        + [pltpu.VMEM((K,), jnp.int32)] * NBUF
        + [pltpu.VMEM((K,), jnp.float32)] * NBUF
        + [pltpu.VMEM_SHARED((NA, D), jnp.float32)]
        + [pltpu.SemaphoreType.DMA] * (4 * NBUF)
    ),
    compiler_params=pltpu.CompilerParams(needs_layout_passes=False),
)
def _edge_kernel(xs_hbm, rc_hbm, w_hbm, out_hbm,
                 b0, b1, b2, rc0, rc1, rc2, ci0, ci1, ci2, w0, w1, w2,
                 t_sp,
                 gs0, gs1, gs2, ss0, ss1, ss2,
                 rs0, rs1, rs2, ws0, ws1, ws2):
    bufs = (b0, b1, b2)
    rcs = (rc0, rc1, rc2)
    cis = (ci0, ci1, ci2)
    wvs = (w0, w1, w2)
    gsems = (gs0, gs1, gs2)
    ssems = (ss0, ss1, ss2)
    rsems = (rs0, rs1, rs2)
    wsems = (ws0, ws1, ws2)

    cid = lax.axis_index("c")
    sid = lax.axis_index("s")
    t = cid * NS + sid
    base = sid * RPT

    def _zrow(i, carry):
        for f in range(8):
            b0[i, pl.ds(f * 16, 16)] = jnp.zeros((16,), jnp.float32)
        return carry
    lax.fori_loop(0, K, _zrow, 0)
    for i in range(5):
        pltpu.sync_copy(b0, t_sp.at[pl.ds(base + i * K, K)])
    pltpu.sync_copy(b0.at[pl.ds(0, RPT - 5 * K)],
                    t_sp.at[pl.ds(base + 5 * K, RPT - 5 * K)])
    plsc.subcore_barrier()

    def _start_stage(j, s):
        pltpu.async_copy(rc_hbm.at[t, j], rcs[s], rsems[s])
        pltpu.async_copy(w_hbm.at[t, j], wvs[s], wsems[s])

    def _wait_stage(j, s):
        pltpu.make_async_copy(rc_hbm.at[t, j], rcs[s], rsems[s]).wait()
        pltpu.make_async_copy(w_hbm.at[t, j], wvs[s], wsems[s]).wait()

    def _unpack(s):
        for g in range(K // 16):
            sl = pl.ds(g * 16, 16)
            v = rcs[s][sl]
            rcs[s][sl] = v & 0x3FFF
            cis[s][sl] = lax.shift_right_logical(v, 14)

    def _start_gather(j, s):
        pltpu.async_copy(xs_hbm.at[rcs[s]], bufs[s], gsems[s])

    def _wait_gather(j, s):
        pltpu.make_async_copy(xs_hbm.at[rcs[s]], bufs[s], gsems[s]).wait()

    def _start_scatter(j, s):
        pltpu.async_copy(bufs[s], t_sp.at[cis[s]], ssems[s], add=True)

    def _wait_scatter(j, s):
        pltpu.make_async_copy(bufs[s], t_sp.at[cis[s]], ssems[s]).wait()

    _start_stage(0, 0)
    _start_stage(1, 1)
    _wait_stage(0, 0)
    _unpack(0)
    _start_gather(0, 0)

    def _group(m, carry):
        for i in range(NBUF):
            j = m * NBUF + i
            s = i
            n = (i + 1) % NBUF
            p = (i + 2) % NBUF

            _wait_gather(j, s)

            def _stage_next():
                _start_stage(j + 2, p)
            if i == 0:
                _stage_next()
            else:
                pl.when(m < GPC - 1)(_stage_next)

            def _prep_next():
                _wait_scatter(j - 2, n)

            def _launch_next():
                _wait_stage(j + 1, n)
                _unpack(n)
                _start_gather(j + 1, n)

            if i == 2:
                _prep_next()
                pl.when(m < GPC - 1)(_launch_next)
            else:
                pl.when(m > 0)(_prep_next)
                _launch_next()

            b = bufs[s]
            wr = wvs[s]

            @plsc.parallel_loop(0, K, unroll=8)
            def _scale(kk):
                wv = plsc.load_gather(wr, [jnp.full((16,), kk, jnp.int32)])
                for f in range(8):
                    sl = pl.ds(f * 16, 16)
                    b[kk, sl] = b[kk, sl] * wv

            _start_scatter(j, s)
        return carry
    lax.fori_loop(0, GPC, _group, 0)

    _wait_scatter(C - 2, (C - 2) % NBUF)
    _wait_scatter(C - 1, (C - 1) % NBUF)

    plsc.subcore_barrier()
    pltpu.sync_copy(t_sp.at[pl.ds(base, RPT)],
                    out_hbm.at[cid, pl.ds(base, RPT)])


def _dis_of(dp):
    deg = 1.0 + jnp.sum(dp, axis=0)[:, None]
    return lax.rsqrt(deg)


def _prep_body(dp_ref, x_ref, xs_ref):
    xs_ref[...] = _dis_of(dp_ref[...]) * x_ref[...]


_prep = pl.pallas_call(
    _prep_body,
    out_shape=jax.ShapeDtypeStruct((N, D), jnp.float32),
)


def _combine_math(tp_ref, h_ref, dp_ref):
    dis = _dis_of(dp_ref[...])
    h = h_ref[...]
    t_sum = tp_ref[0, :N, :] + tp_ref[1, :N, :]
    agg = dis * t_sum + dis * dis * h
    o = jax.nn.sigmoid((1.0 - ALPHA_C) * agg + ALPHA_C * h)
    return dis, o


def _combine_body(tp_ref, h_ref, dp_ref, out_ref, xs_ref):
    dis, o = _combine_math(tp_ref, h_ref, dp_ref)
    out_ref[...] = o
    xs_ref[...] = dis * o


_combine = pl.pallas_call(
    _combine_body,
    out_shape=[
        jax.ShapeDtypeStruct((N, D), jnp.float32),
        jax.ShapeDtypeStruct((N, D), jnp.float32),
    ],
)


def _combine_last_body(tp_ref, h_ref, dp_ref, out_ref):
    _, o = _combine_math(tp_ref, h_ref, dp_ref)
    out_ref[...] = o


_combine_last = pl.pallas_call(
    _combine_last_body,
    out_shape=jax.ShapeDtypeStruct((N, D), jnp.float32),
)


def _prepare_edges(edge_index, edge_weight):
    row = edge_index[0].astype(jnp.int32).reshape(NT, EPT)
    col = edge_index[1].astype(jnp.int32).reshape(NT, EPT)
    w = edge_weight.astype(jnp.float32).reshape(NT, EPT)

    spread = (jnp.arange(PAD, dtype=jnp.int32) * 41) % N
    pad_idx = jnp.broadcast_to(spread, (NT, PAD))
    pad_w = jnp.zeros((NT, PAD), jnp.float32)
    row_p = jnp.concatenate([row, pad_idx], axis=1)
    col_p = jnp.concatenate([col, pad_idx], axis=1)
    w_p = jnp.concatenate([w, pad_w], axis=1)
    rc_p = (row_p | (col_p << 14)).reshape(NT, C, K)
    return rc_p, col_p, w_p


def kernel(x, edge_index, edge_weight):
    x = x.astype(jnp.float32)
    rc_p, col_p, w_p = _prepare_edges(edge_index, edge_weight)
    w_ck = w_p.reshape(NT, C, K)

    deg_parts = _deg_kernel(col_p, w_p)
    xs1 = _prep(deg_parts, x)
    t_parts = _edge_kernel(xs1, rc_p, w_ck)
    h, xs2 = _combine(t_parts, x, deg_parts)
    u_parts = _edge_kernel(xs2, rc_p, w_ck)
    return _combine_last(u_parts, h, deg_parts)
